# baseline (speedup 1.0000x reference)
import jax
import jax.numpy as jnp
from jax.experimental import pallas as pl


def _source_kernel(w_ref, ctx_ref, out_ref):
    out_ref[0] = jnp.dot(w_ref[...], ctx_ref[0],
                         preferred_element_type=jnp.float32)


def _attn_kernel(x_ref, seg_ref, s_ref, wc_ref, attn_ref):
    x = x_ref[0]
    s = s_ref[0]
    seg = seg_ref[0]
    a = jax.lax.dot_general(s, x, (((0,), (0,)), ((), ())),
                            preferred_element_type=jnp.float32)
    z = jnp.any(seg != 0.0, axis=0, keepdims=True)
    m = jnp.max(a, axis=0, keepdims=True)
    e = jnp.exp(a - m)
    sm = e / jnp.sum(e, axis=0, keepdims=True)
    attn = jnp.where(z, sm, 0.0)
    attn_ref[0] = attn
    wc_ref[0] = jnp.dot(s, attn, preferred_element_type=jnp.float32)


def kernel(input, context, seg, W_ctx, opt=0):
    B, idf, ih, iw = input.shape
    queryL = ih * iw
    cdf, sourceL = context.shape[1], context.shape[2]
    p_num = seg.shape[1]

    x = input.reshape(B, idf, queryL)
    segf = seg.reshape(B, p_num, queryL)

    sourceT = pl.pallas_call(
        _source_kernel,
        grid=(B,),
        in_specs=[
            pl.BlockSpec((idf, cdf), lambda b: (0, 0)),
            pl.BlockSpec((1, cdf, sourceL), lambda b: (b, 0, 0)),
        ],
        out_specs=pl.BlockSpec((1, idf, sourceL), lambda b: (b, 0, 0)),
        out_shape=jax.ShapeDtypeStruct((B, idf, sourceL), jnp.float32),
    )(W_ctx, context)

    Qt = 512
    nq = queryL // Qt
    wc, attn = pl.pallas_call(
        _attn_kernel,
        grid=(B, nq),
        in_specs=[
            pl.BlockSpec((1, idf, Qt), lambda b, q: (b, 0, q)),
            pl.BlockSpec((1, p_num, Qt), lambda b, q: (b, 0, q)),
            pl.BlockSpec((1, idf, sourceL), lambda b, q: (b, 0, 0)),
        ],
        out_specs=[
            pl.BlockSpec((1, idf, Qt), lambda b, q: (b, 0, q)),
            pl.BlockSpec((1, sourceL, Qt), lambda b, q: (b, 0, q)),
        ],
        out_shape=[
            jax.ShapeDtypeStruct((B, idf, queryL), jnp.float32),
            jax.ShapeDtypeStruct((B, sourceL, queryL), jnp.float32),
        ],
    )(x, segf, sourceT)

    return (wc.reshape(B, idf, ih, iw), attn.reshape(B, sourceL, ih, iw))


# speedup vs baseline: 1.1652x; 1.1652x over previous
import jax
import jax.numpy as jnp
from jax.experimental import pallas as pl
from jax.experimental.pallas import tpu as pltpu


def _attn_kernel(w_ref, ctx_ref, x_ref, seg_ref, wc_ref, attn_ref, s_ref):
    q_idx = pl.program_id(1)

    @pl.when(q_idx == 0)
    def _():
        s_ref[...] = jnp.dot(w_ref[...], ctx_ref[0],
                             preferred_element_type=jnp.float32)

    x = x_ref[0]
    s = s_ref[...]
    seg = seg_ref[0]
    a = jax.lax.dot_general(s, x, (((0,), (0,)), ((), ())),
                            preferred_element_type=jnp.float32)
    z = jnp.any(seg != 0.0, axis=0, keepdims=True)
    m = jnp.max(a, axis=0, keepdims=True)
    e = jnp.exp(a - m)
    sm = e / jnp.sum(e, axis=0, keepdims=True)
    attn = jnp.where(z, sm, 0.0)
    attn_ref[0] = attn
    wc_ref[0] = jnp.dot(s, attn, preferred_element_type=jnp.float32)


def kernel(input, context, seg, W_ctx, opt=0):
    B, idf, ih, iw = input.shape
    queryL = ih * iw
    cdf, sourceL = context.shape[1], context.shape[2]
    p_num = seg.shape[1]

    x = input.reshape(B, idf, queryL)
    segf = seg.reshape(B, p_num, queryL)

    Qt = 1024
    nq = queryL // Qt
    wc, attn = pl.pallas_call(
        _attn_kernel,
        grid=(B, nq),
        in_specs=[
            pl.BlockSpec((idf, cdf), lambda b, q: (0, 0)),
            pl.BlockSpec((1, cdf, sourceL), lambda b, q: (b, 0, 0)),
            pl.BlockSpec((1, idf, Qt), lambda b, q: (b, 0, q)),
            pl.BlockSpec((1, p_num, Qt), lambda b, q: (b, 0, q)),
        ],
        out_specs=[
            pl.BlockSpec((1, idf, Qt), lambda b, q: (b, 0, q)),
            pl.BlockSpec((1, sourceL, Qt), lambda b, q: (b, 0, q)),
        ],
        out_shape=[
            jax.ShapeDtypeStruct((B, idf, queryL), jnp.float32),
            jax.ShapeDtypeStruct((B, sourceL, queryL), jnp.float32),
        ],
        scratch_shapes=[pltpu.VMEM((idf, sourceL), jnp.float32)],
        compiler_params=pltpu.CompilerParams(
            dimension_semantics=("parallel", "arbitrary")),
    )(W_ctx, context, x, segf)

    return (wc.reshape(B, idf, ih, iw), attn.reshape(B, sourceL, ih, iw))


# speedup vs baseline: 1.2352x; 1.0601x over previous
import jax
import jax.numpy as jnp
from jax.experimental import pallas as pl
from jax.experimental.pallas import tpu as pltpu


def _attn_kernel(w_ref, ctx_ref, x_ref, seg_ref, wc_ref, attn_ref, s_ref):
    q_idx = pl.program_id(1)

    @pl.when(q_idx == 0)
    def _():
        s_ref[...] = jnp.dot(w_ref[...], ctx_ref[0],
                             preferred_element_type=jnp.float32)

    x = x_ref[0]
    s = s_ref[...]
    seg = seg_ref[0]
    a = jax.lax.dot_general(s, x, (((0,), (0,)), ((), ())),
                            preferred_element_type=jnp.float32)
    z = jnp.any(seg != 0.0, axis=0, keepdims=True)
    m = jnp.max(a, axis=0, keepdims=True)
    e = jnp.exp(a - m)
    sm = e / jnp.sum(e, axis=0, keepdims=True)
    attn = jnp.where(z, sm, 0.0)
    attn_ref[0] = attn
    wc_ref[0] = jnp.dot(s, attn, preferred_element_type=jnp.float32)


def kernel(input, context, seg, W_ctx, opt=0):
    B, idf, ih, iw = input.shape
    queryL = ih * iw
    cdf, sourceL = context.shape[1], context.shape[2]
    p_num = seg.shape[1]

    x = input.reshape(B, idf, queryL)
    segf = seg.reshape(B, p_num, queryL)

    Qt = 2048
    nq = queryL // Qt
    wc, attn = pl.pallas_call(
        _attn_kernel,
        grid=(B, nq),
        in_specs=[
            pl.BlockSpec((idf, cdf), lambda b, q: (0, 0)),
            pl.BlockSpec((1, cdf, sourceL), lambda b, q: (b, 0, 0)),
            pl.BlockSpec((1, idf, Qt), lambda b, q: (b, 0, q)),
            pl.BlockSpec((1, p_num, Qt), lambda b, q: (b, 0, q)),
        ],
        out_specs=[
            pl.BlockSpec((1, idf, Qt), lambda b, q: (b, 0, q)),
            pl.BlockSpec((1, sourceL, Qt), lambda b, q: (b, 0, q)),
        ],
        out_shape=[
            jax.ShapeDtypeStruct((B, idf, queryL), jnp.float32),
            jax.ShapeDtypeStruct((B, sourceL, queryL), jnp.float32),
        ],
        scratch_shapes=[pltpu.VMEM((idf, sourceL), jnp.float32)],
        compiler_params=pltpu.CompilerParams(
            dimension_semantics=("parallel", "arbitrary")),
    )(W_ctx, context, x, segf)

    return (wc.reshape(B, idf, ih, iw), attn.reshape(B, sourceL, ih, iw))


# speedup vs baseline: 1.2445x; 1.0075x over previous
import jax
import jax.numpy as jnp
from jax.experimental import pallas as pl
from jax.experimental.pallas import tpu as pltpu


def _attn_kernel(w_ref, ctx_ref, x_ref, seg_ref, wc_ref, attn_ref, s_ref):
    q_idx = pl.program_id(1)

    @pl.when(q_idx == 0)
    def _():
        s_ref[...] = jnp.dot(w_ref[...], ctx_ref[0],
                             preferred_element_type=jnp.float32)

    x = x_ref[0]
    s = s_ref[...]
    seg = seg_ref[0]
    a = jax.lax.dot_general(s, x, (((0,), (0,)), ((), ())),
                            preferred_element_type=jnp.float32)
    z = jnp.any(seg != 0.0, axis=0, keepdims=True)
    m = jnp.max(a, axis=0, keepdims=True)
    e = jnp.exp(a - m)
    sm = e / jnp.sum(e, axis=0, keepdims=True)
    attn = jnp.where(z, sm, 0.0)
    attn_ref[0] = attn
    wc_ref[0] = jnp.dot(s, attn, preferred_element_type=jnp.float32)


def kernel(input, context, seg, W_ctx, opt=0):
    B, idf, ih, iw = input.shape
    queryL = ih * iw
    cdf, sourceL = context.shape[1], context.shape[2]
    p_num = seg.shape[1]

    x = input.reshape(B, idf, queryL)
    segf = seg.reshape(B, p_num, queryL)

    Qt = 4096
    nq = queryL // Qt
    wc, attn = pl.pallas_call(
        _attn_kernel,
        grid=(B, nq),
        in_specs=[
            pl.BlockSpec((idf, cdf), lambda b, q: (0, 0)),
            pl.BlockSpec((1, cdf, sourceL), lambda b, q: (b, 0, 0)),
            pl.BlockSpec((1, idf, Qt), lambda b, q: (b, 0, q)),
            pl.BlockSpec((1, p_num, Qt), lambda b, q: (b, 0, q)),
        ],
        out_specs=[
            pl.BlockSpec((1, idf, Qt), lambda b, q: (b, 0, q)),
            pl.BlockSpec((1, sourceL, Qt), lambda b, q: (b, 0, q)),
        ],
        out_shape=[
            jax.ShapeDtypeStruct((B, idf, queryL), jnp.float32),
            jax.ShapeDtypeStruct((B, sourceL, queryL), jnp.float32),
        ],
        scratch_shapes=[pltpu.VMEM((idf, sourceL), jnp.float32)],
        compiler_params=pltpu.CompilerParams(
            dimension_semantics=("parallel", "arbitrary")),
    )(W_ctx, context, x, segf)

    return (wc.reshape(B, idf, ih, iw), attn.reshape(B, sourceL, ih, iw))
